# baseline (speedup 1.0000x reference)
import math

import jax
import jax.numpy as jnp
from jax import lax
from jax.experimental import pallas as pl
from jax.experimental.pallas import tpu as pltpu
from jax.experimental.pallas import tpu_sc as plsc

N = 10000
F = 128
H = 128
C = 40
E = 320000
K = 20
NP = 10112
RPT = NP // 16
CH = 112
EPT = 21504
EPAD = EPT * 16
NG = 12
FW = 128
_CHUNKS = [(i * CH, min(CH, RPT - i * CH)) for i in range((RPT + CH - 1) // CH)]


def _coef_table():
    import numpy as np
    t = np.zeros((16, 128), np.float32)
    for k in range(K + 1):
        if k % 2 == 0:
            s = float((-1) ** (k // 2))
            co = 48 * (k // 16)
            t[k % 16, co + 0:co + 16] = s * 3.0 ** k / math.factorial(k)
            t[k % 16, co + 16:co + 32] = s * 2.0 ** k / math.factorial(k)
            t[k % 16, co + 32:co + 48] = s * 1.0 ** k / math.factorial(k)
    return jnp.asarray(t)


def _sc_deg_body(dstp, deg_out, dbuf, onesb, stg, degslab):
    c = lax.axis_index("c")
    w = lax.axis_index("s")
    row0 = w * RPT
    zero16 = jnp.zeros((16,), jnp.float32)
    one16 = jnp.ones((16,), jnp.float32)

    def _fill(i, _):
        for v in range(8):
            onesb[i, pl.ds(16 * v, 16)] = one16
            stg[i, pl.ds(16 * v, 16)] = zero16
        return 0
    lax.fori_loop(0, 128, _fill, 0)

    for base, nr in _CHUNKS:
        pltpu.sync_copy(stg.at[pl.ds(0, nr)],
                        degslab.at[pl.ds(row0 + base, nr)])
    plsc.subcore_barrier()

    def _group(g, _):
        gg = (w * NG + g) * 32 + 16
        pltpu.sync_copy(dstp.at[pl.ds(gg, 16)], dbuf)

        def _chunk(j, _):
            pltpu.sync_copy(onesb.at[pl.ds(0, CH)], degslab.at[dbuf.at[j]],
                            add=True)
            return 0
        lax.fori_loop(0, 16, _chunk, 0)
        return 0
    lax.fori_loop(0, NG, _group, 0)
    plsc.subcore_barrier()

    @pl.when(c == 0)
    def _():
        for base, nr in _CHUNKS:
            pltpu.sync_copy(degslab.at[pl.ds(row0 + base, nr)],
                            stg.at[pl.ds(0, nr)])
            pltpu.sync_copy(stg.at[pl.ds(0, nr)],
                            deg_out.at[pl.ds(row0 + base, nr)])


def _sc_deg(dstp):
    mesh = plsc.VectorSubcoreMesh(core_axis_name="c", subcore_axis_name="s")
    return pl.kernel(
        _sc_deg_body,
        out_type=jax.ShapeDtypeStruct((NP, FW), jnp.float32),
        mesh=mesh,
        scratch_types=[
            pltpu.VMEM((16, CH), jnp.int32),
            pltpu.VMEM((128, FW), jnp.float32),
            pltpu.VMEM((128, FW), jnp.float32),
            pltpu.VMEM_SHARED((NP, FW), jnp.float32),
        ],
    )(dstp)


def _tc1_body(x_ref, w0_ref, b0_ref, deg_ref, us_ref, acc_ref):
    z = lax.dot_general(x_ref[...], w0_ref[...], (((1,), (1,)), ((), ())),
                        preferred_element_type=jnp.float32) + b0_ref[...]
    d = jnp.maximum(deg_ref[:, 0:1], 1.0)
    dinv = lax.rsqrt(d)
    rdeg = jnp.broadcast_to(1.0 / d, (NP, 16))
    dinvr = jnp.broadcast_to(dinv, (NP, 16))
    zeros48 = jnp.zeros((NP, 48), jnp.float32)
    zeros16 = jnp.zeros((NP, 16), jnp.float32)
    for h, sl in ((0, slice(0, 64)), (1, slice(64, 128))):
        zh = z[:, sl] * dinv
        us_ref[h * NP:(h + 1) * NP, 0:64] = zh
        us_ref[h * NP:(h + 1) * NP, 64:80] = dinvr
        us_ref[h * NP:(h + 1) * NP, 80:128] = zeros48
        acc_ref[h * NP:(h + 1) * NP, 0:64] = zh
        acc_ref[h * NP:(h + 1) * NP, 64:80] = dinvr
        acc_ref[h * NP:(h + 1) * NP, 80:96] = dinvr
        acc_ref[h * NP:(h + 1) * NP, 96:112] = rdeg
        acc_ref[h * NP:(h + 1) * NP, 112:128] = zeros16


def _tc1(x_pad, W0, b0r, degrep):
    full = lambda shape: pl.BlockSpec(shape, lambda: tuple(0 for _ in shape))
    return pl.pallas_call(
        _tc1_body,
        in_specs=[full((NP, F)), full((H, F)), full((1, H)), full((NP, FW))],
        out_specs=[full((2 * NP, FW)), full((2 * NP, FW))],
        out_shape=[
            jax.ShapeDtypeStruct((2 * NP, FW), jnp.float32),
            jax.ShapeDtypeStruct((2 * NP, FW), jnp.float32),
        ],
    )(x_pad, W0, b0r, degrep)


def _sc_body(usinit, accinit, ixp, ctab,
             acc_out, us,
             ibuf, b0, b1, b2, ctb,
             outslab, sem, sg0, sg1, sg2, ss0, ss1, ss2):
    c = lax.axis_index("c")
    w = lax.axis_index("s")
    row0 = w * RPT
    zero16 = jnp.zeros((16,), jnp.float32)

    pltpu.sync_copy(ctab, ctb)

    for base, nr in _CHUNKS:
        pltpu.sync_copy(usinit.at[pl.ds(c * NP + row0 + base, nr)],
                        b0.at[pl.ds(0, nr)])
        pltpu.sync_copy(b0.at[pl.ds(0, nr)],
                        us.at[pl.ds(c * NP + row0 + base, nr)])
        pltpu.sync_copy(accinit.at[pl.ds(c * NP + row0 + base, nr)],
                        b1.at[pl.ds(0, nr)])
        pltpu.sync_copy(b1.at[pl.ds(0, nr)],
                        acc_out.at[pl.ds(c * NP + row0 + base, nr)])

    def _prop(k, _):
        def _zrow(i, _):
            for v in range(8):
                b0[i, pl.ds(16 * v, 16)] = zero16
            return 0
        lax.fori_loop(0, CH, _zrow, 0)
        for base, nr in _CHUNKS:
            pltpu.async_copy(b0.at[pl.ds(0, nr)],
                             outslab.at[pl.ds(row0 + base, nr)], sem)
        for base, nr in _CHUNKS:
            pltpu.make_async_copy(b0.at[pl.ds(0, nr)],
                                  outslab.at[pl.ds(row0 + base, nr)], sem).wait()
        plsc.subcore_barrier()

        def _g(sidx, j, buf, gsem):
            pltpu.async_copy(us.at[sidx.at[j]], buf, gsem)

        def _wg(sidx, buf, gsem):
            pltpu.make_async_copy(us.at[sidx.at[0]], buf, gsem).wait()

        def _s(didx, j, buf, ssem):
            pltpu.async_copy(buf, outslab.at[didx.at[16 + j]], ssem, add=True)

        def _ws(didx, buf, ssem):
            pltpu.make_async_copy(buf, outslab.at[didx.at[16]], ssem).wait()

        coff = c * NP

        def _group(g, _):
            gg = (w * NG + g) * 32
            pltpu.sync_copy(ixp.at[pl.ds(gg, 32)], ibuf)

            def _adjrow(i, _):
                for v in range(7):
                    ibuf[i, pl.ds(16 * v, 16)] = ibuf[i, pl.ds(16 * v, 16)] + coff
                return 0
            lax.fori_loop(0, 16, _adjrow, 0)

            _g(ibuf, 0, b0, sg0)
            _g(ibuf, 1, b1, sg1)
            _wg(ibuf, b0, sg0)
            _s(ibuf, 0, b0, ss0)
            _g(ibuf, 2, b2, sg2)
            _wg(ibuf, b1, sg1)
            _s(ibuf, 1, b1, ss1)
            _ws(ibuf, b0, ss0)
            _g(ibuf, 3, b0, sg0)

            def _tri(t, _):
                j = 3 * t + 2
                _wg(ibuf, b2, sg2)
                _s(ibuf, j, b2, ss2)
                _ws(ibuf, b1, ss1)
                _g(ibuf, j + 2, b1, sg1)
                _wg(ibuf, b0, sg0)
                _s(ibuf, j + 1, b0, ss0)
                _ws(ibuf, b2, ss2)
                _g(ibuf, j + 3, b2, sg2)
                _wg(ibuf, b1, sg1)
                _s(ibuf, j + 2, b1, ss1)
                _ws(ibuf, b0, ss0)
                _g(ibuf, j + 4, b0, sg0)
                return 0
            lax.fori_loop(0, 4, _tri, 0)
            _wg(ibuf, b2, sg2)
            _s(ibuf, 14, b2, ss2)
            _ws(ibuf, b1, ss1)
            _wg(ibuf, b0, sg0)
            _s(ibuf, 15, b0, ss0)
            _ws(ibuf, b2, ss2)
            _ws(ibuf, b0, ss0)
            return 0
        lax.fori_loop(0, NG, _group, 0)
        plsc.subcore_barrier()

        rowk = k % 16
        co = (k // 16) * 48
        c3v = ctb[rowk, pl.ds(co, 16)]
        c2v = ctb[rowk, pl.ds(co + 16, 16)]
        c1v = ctb[rowk, pl.ds(co + 32, 16)]
        for base, nr in _CHUNKS:
            pltpu.async_copy(outslab.at[pl.ds(row0 + base, nr)],
                             b0.at[pl.ds(0, nr)], sg0)
            pltpu.async_copy(acc_out.at[pl.ds(c * NP + row0 + base, nr)],
                             b1.at[pl.ds(0, nr)], sg1)
            pltpu.make_async_copy(outslab.at[pl.ds(row0 + base, nr)],
                                  b0.at[pl.ds(0, nr)], sg0).wait()
            pltpu.make_async_copy(acc_out.at[pl.ds(c * NP + row0 + base, nr)],
                                  b1.at[pl.ds(0, nr)], sg1).wait()

            def _post_row(r, _):
                rv = b1[r, pl.ds(96, 16)]
                for v in range(4):
                    t = b0[r, pl.ds(16 * v, 16)] * rv
                    b0[r, pl.ds(16 * v, 16)] = t
                    b1[r, pl.ds(16 * v, 16)] = b1[r, pl.ds(16 * v, 16)] + c3v * t
                t = b0[r, pl.ds(64, 16)] * rv
                b0[r, pl.ds(64, 16)] = t
                b1[r, pl.ds(64, 16)] = b1[r, pl.ds(64, 16)] + c2v * t
                b1[r, pl.ds(80, 16)] = b1[r, pl.ds(80, 16)] + c1v * t
                return 0
            lax.fori_loop(0, nr, _post_row, 0)
            pltpu.async_copy(b0.at[pl.ds(0, nr)],
                             us.at[pl.ds(c * NP + row0 + base, nr)], ss0)
            pltpu.async_copy(b1.at[pl.ds(0, nr)],
                             acc_out.at[pl.ds(c * NP + row0 + base, nr)], ss1)
            pltpu.make_async_copy(b0.at[pl.ds(0, nr)],
                                  us.at[pl.ds(c * NP + row0 + base, nr)], ss0).wait()
            pltpu.make_async_copy(b1.at[pl.ds(0, nr)],
                                  acc_out.at[pl.ds(c * NP + row0 + base, nr)], ss1).wait()
        return 0
    lax.fori_loop(1, K + 1, _prop, 0)


def _sc_call(usinit, accinit, ixp, ctab):
    mesh = plsc.VectorSubcoreMesh(core_axis_name="c", subcore_axis_name="s")
    f32 = jnp.float32
    return pl.kernel(
        _sc_body,
        out_type=(
            jax.ShapeDtypeStruct((2 * NP, FW), f32),
            jax.ShapeDtypeStruct((2 * NP, FW), f32),
        ),
        mesh=mesh,
        scratch_types=[
            pltpu.VMEM((32, CH), jnp.int32),
            pltpu.VMEM((CH, FW), f32),
            pltpu.VMEM((CH, FW), f32),
            pltpu.VMEM((CH, FW), f32),
            pltpu.VMEM((16, 128), f32),
            pltpu.VMEM_SHARED((NP, FW), f32),
            pltpu.SemaphoreType.DMA,
            pltpu.SemaphoreType.DMA,
            pltpu.SemaphoreType.DMA,
            pltpu.SemaphoreType.DMA,
            pltpu.SemaphoreType.DMA,
            pltpu.SemaphoreType.DMA,
            pltpu.SemaphoreType.DMA,
        ],
    )(usinit, accinit, ixp, ctab)


def _tc2_body(a0_ref, a1_ref, W1_ref, W2_ref, b1_ref, b2_ref,
              Wm0_ref, bm0_ref, Wm1_ref, bm1_ref, Wm2_ref, bm2_ref,
              Wm3_ref, bm3_ref, out_ref):
    dg = lambda a, b: lax.dot_general(a, b, (((1,), (1,)), ((), ())),
                                      preferred_element_type=jnp.float32)
    a0 = a0_ref[...]
    a1 = a1_ref[...]
    sq = lax.rsqrt(a0[:, 96:97])
    S = jnp.concatenate([a0[:, 0:64], a1[:, 0:64]], axis=1) * sq
    h = dg(dg(S, W1_ref[...]), W2_ref[...])
    w2b1 = dg(b1_ref[...], W2_ref[...])
    h = h + (a0[:, 64:65] * sq) * w2b1 + (a0[:, 80:81] * sq) * b2_ref[...]
    h = jnp.maximum(dg(h, Wm0_ref[...]) + bm0_ref[...], 0.0)
    h = jnp.maximum(dg(h, Wm1_ref[...]) + bm1_ref[...], 0.0)
    h = jnp.maximum(dg(h, Wm2_ref[...]) + bm2_ref[...], 0.0)
    lg = dg(h, Wm3_ref[...]) + bm3_ref[...]
    m = jnp.max(lg, axis=1, keepdims=True)
    s = jnp.sum(jnp.exp(lg - m), axis=1, keepdims=True)
    out_ref[...] = lg - m - jnp.log(s)


def _tc2(a0, a1, W1, W2, b1r, b2r, Wm0, bm0r, Wm1, bm1r, Wm2, bm2r, Wm3, bm3r):
    full = lambda shape: pl.BlockSpec(shape, lambda i: tuple(0 for _ in shape))
    return pl.pallas_call(
        _tc2_body,
        grid=(25,),
        in_specs=[
            pl.BlockSpec((400, FW), lambda i: (i, 0)),
            pl.BlockSpec((400, FW), lambda i: (i, 0)),
            full((H, H)), full((H, H)), full((1, H)), full((1, H)),
            full((H, H)), full((1, H)), full((H, H)), full((1, H)),
            full((H, H)), full((1, H)), full((C, H)), full((1, C)),
        ],
        out_specs=pl.BlockSpec((400, C), lambda i: (i, 0)),
        out_shape=jax.ShapeDtypeStruct((N, C), jnp.float32),
    )(a0, a1, W1, W2, b1r, b2r, Wm0, bm0r, Wm1, bm1r, Wm2, bm2r, Wm3, bm3r)


def kernel(x_in, edge_index, Wc0, bc0, Wc1, bc1, Wc2, bc2,
           Wm0, bm0, Wm1, bm1, Wm2, bm2, Wm3, bm3):
    f32 = jnp.float32
    src = edge_index[0]
    dst = edge_index[1]
    npad = EPAD - E
    pad_src = (jnp.arange(npad, dtype=jnp.int32) * 1009) % N
    pad_dst = N + (jnp.arange(npad, dtype=jnp.int32) % 16)
    srcp = jnp.concatenate([src, pad_src]).reshape(EPAD // (16 * CH), 16, CH)
    dstp = jnp.concatenate([dst, pad_dst]).reshape(EPAD // (16 * CH), 16, CH)
    ixp = jnp.concatenate([srcp, dstp], axis=1).reshape(2 * EPAD // CH, CH)

    degrep = _sc_deg(ixp)

    x_pad = jnp.concatenate([x_in, jnp.zeros((NP - N, F), f32)], axis=0)
    usinit, accinit = _tc1(x_pad, Wc0, bc0.reshape(1, H), degrep)

    ctab = _coef_table()
    acc, _ = _sc_call(usinit, accinit, ixp, ctab)

    out = _tc2(acc[0:NP], acc[NP:2 * NP],
               Wc1, Wc2, bc1.reshape(1, H), bc2.reshape(1, H),
               Wm0, bm0.reshape(1, H), Wm1, bm1.reshape(1, H),
               Wm2, bm2.reshape(1, H), Wm3, bm3.reshape(1, C))
    return out
